# baseline (speedup 1.0000x reference)
import functools

import jax
import jax.numpy as jnp
from jax import lax
from jax.experimental import pallas as pl
from jax.experimental.pallas import tpu as pltpu
from jax.experimental.pallas import tpu_sc as plsc

D = 5120
FF = 384
NE = 32
NS = 2
TOPK = 2
T = 2048

TB = 256
NTB = T // TB
BM = 128
NP = T * TOPK
NBMAX = NP // BM + NE
PPAD = NBMAX * BM

NW = 32
TPW = T // NW
CH = 8
NCH = TPW // CH
FF2 = NS * FF

_INV_SQRT2 = 0.7071067811865476


def _erf(x):
    s = jnp.sign(x)
    a = jnp.abs(x)
    t = 1.0 / (1.0 + 0.3275911 * a)
    poly = t * (0.254829592 + t * (-0.284496736 + t * (1.421413741 + t * (
        -1.453152027 + t * 1.061405429))))
    return s * (1.0 - poly * jnp.exp(-a * a))


def _gelu(x):
    return 0.5 * x * (1.0 + _erf(x * _INV_SQRT2))


def _gate_shared_body(xb, gw, gb, w1, b1, w2, b2,
                      oh0, oh1, p0, p1, so):
    xv = xb[...]
    logits = jnp.dot(xv, gw[...], preferred_element_type=jnp.float32) + gb[...]
    m = jnp.max(logits, axis=1, keepdims=True)
    ex = jnp.exp(logits - m)
    probs = ex / jnp.sum(ex, axis=1, keepdims=True)
    lane = lax.broadcasted_iota(jnp.int32, (TB, NE), 1)
    m1 = jnp.max(probs, axis=1, keepdims=True)
    a1 = jnp.min(jnp.where(probs == m1, lane, NE), axis=1, keepdims=True)
    sel1 = lane == a1
    pm = jnp.where(sel1, -jnp.inf, probs)
    m2 = jnp.max(pm, axis=1, keepdims=True)
    a2 = jnp.min(jnp.where(pm == m2, lane, NE), axis=1, keepdims=True)
    sel2 = lane == a2
    oh0[...] = sel1.astype(jnp.float32)
    oh1[...] = sel2.astype(jnp.float32)
    p0[...] = m1
    p1[...] = m2

    h = jnp.dot(xv, w1[...], preferred_element_type=jnp.float32) + b1[...]
    h = _gelu(h)
    so[...] = jnp.dot(h, w2[...], preferred_element_type=jnp.float32) + b2[...]


def _gate_shared(x, gate_w, gb, w1c, b1c, w2c, b2c):
    return pl.pallas_call(
        _gate_shared_body,
        grid=(NTB,),
        in_specs=[
            pl.BlockSpec((TB, D), lambda t: (t, 0)),
            pl.BlockSpec((D, NE), lambda t: (0, 0)),
            pl.BlockSpec((1, NE), lambda t: (0, 0)),
            pl.BlockSpec((D, FF2), lambda t: (0, 0)),
            pl.BlockSpec((1, FF2), lambda t: (0, 0)),
            pl.BlockSpec((FF2, D), lambda t: (0, 0)),
            pl.BlockSpec((1, D), lambda t: (0, 0)),
        ],
        out_specs=[
            pl.BlockSpec((TB, NE), lambda t: (t, 0)),
            pl.BlockSpec((TB, NE), lambda t: (t, 0)),
            pl.BlockSpec((TB, 1), lambda t: (t, 0)),
            pl.BlockSpec((TB, 1), lambda t: (t, 0)),
            pl.BlockSpec((TB, D), lambda t: (t, 0)),
        ],
        out_shape=[
            jax.ShapeDtypeStruct((T, NE), jnp.float32),
            jax.ShapeDtypeStruct((T, NE), jnp.float32),
            jax.ShapeDtypeStruct((T, 1), jnp.float32),
            jax.ShapeDtypeStruct((T, 1), jnp.float32),
            jax.ShapeDtypeStruct((T, D), jnp.float32),
        ],
        compiler_params=pltpu.CompilerParams(
            dimension_semantics=("arbitrary",)),
    )(x, gate_w, gb, w1c, b1c, w2c, b2c)


def _cumsum0(a, n):
    s = 1
    while s < n:
        shifted = jnp.concatenate(
            [jnp.zeros((s, NE), jnp.float32), a[:-s, :]], axis=0)
        a = a + shifted
        s *= 2
    return a


def _plan_body(oh0_ref, oh1_ref, slot_ref, s2_ref, be_ref, nact_ref):
    oh0 = oh0_ref[...]
    oh1 = oh1_ref[...]
    cum0 = _cumsum0(oh0, T)
    cum1 = _cumsum0(oh1, T)
    tot0 = cum0[T - 1:T, :]
    tot1 = cum1[T - 1:T, :]
    cnt = tot0 + tot1
    nb = jnp.floor((cnt + (BM - 1)) * (1.0 / BM))
    r = lax.broadcasted_iota(jnp.int32, (NE, NE), 0)
    c = lax.broadcasted_iota(jnp.int32, (NE, NE), 1)
    ltri = (r < c).astype(jnp.float32)
    bstart = jnp.dot(nb, ltri, preferred_element_type=jnp.float32)
    nact = jnp.sum(nb, axis=1, keepdims=True)
    start_rows = bstart * BM
    slot0 = jnp.sum(oh0 * (start_rows + cum0 - 1.0), axis=1, keepdims=True)
    slot1 = jnp.sum(oh1 * (start_rows + tot0 + cum1 - 1.0), axis=1,
                    keepdims=True)
    slot_ref[0:T, :] = slot0.astype(jnp.int32)
    slot_ref[T:NP, :] = slot1.astype(jnp.int32)
    s2_ref[0:T, :] = (2.0 * slot0).astype(jnp.int32)
    s2_ref[T:NP, :] = (2.0 * slot1).astype(jnp.int32)
    s2_ref[NP:NP + T, :] = (2.0 * slot0 + 1.0).astype(jnp.int32)
    s2_ref[NP + T:2 * NP, :] = (2.0 * slot1 + 1.0).astype(jnp.int32)
    bio = lax.broadcasted_iota(jnp.int32, (NBMAX, 1), 0).astype(jnp.float32)
    bclamp = jnp.minimum(bio, nact - 1.0)
    be = jnp.sum(jnp.where(bstart <= bclamp, 1.0, 0.0), axis=1,
                 keepdims=True) - 1.0
    be_ref[...] = be.astype(jnp.int32)
    nact_ref[...] = nact.astype(jnp.int32)


def _plan(oh0, oh1):
    return pl.pallas_call(
        _plan_body,
        grid=(1,),
        in_specs=[
            pl.BlockSpec((T, NE), lambda i: (0, 0)),
            pl.BlockSpec((T, NE), lambda i: (0, 0)),
        ],
        out_specs=[
            pl.BlockSpec((NP, 1), lambda i: (0, 0)),
            pl.BlockSpec((2 * NP, 1), lambda i: (0, 0)),
            pl.BlockSpec((NBMAX, 1), lambda i: (0, 0)),
            pl.BlockSpec((1, 1), lambda i: (0, 0)),
        ],
        out_shape=[
            jax.ShapeDtypeStruct((NP, 1), jnp.int32),
            jax.ShapeDtypeStruct((2 * NP, 1), jnp.int32),
            jax.ShapeDtypeStruct((NBMAX, 1), jnp.int32),
            jax.ShapeDtypeStruct((1, 1), jnp.int32),
        ],
    )(oh0, oh1)


def _sc_mesh():
    return plsc.VectorSubcoreMesh(core_axis_name="c", subcore_axis_name="s",
                                  num_cores=2, num_subcores=16)


@functools.lru_cache(maxsize=None)
def _build_sc_dispatch():
    return functools.partial(
        pl.kernel,
        mesh=_sc_mesh(),
        out_type=jax.ShapeDtypeStruct((PPAD, D), jnp.float32),
        scratch_types=[
            pltpu.VMEM((CH, D), jnp.float32),
            pltpu.VMEM((CH,), jnp.int32),
            pltpu.VMEM((CH,), jnp.int32),
            pltpu.SemaphoreType.DMA,
            pltpu.SemaphoreType.DMA,
        ],
    )(_sc_dispatch_body)


def _sc_dispatch_body(x_hbm, slot_hbm, xg_hbm, rows_v, idx0_v, idx1_v, s0, s1):
    wid = lax.axis_index("s") * 2 + lax.axis_index("c")
    base = wid * TPW

    def chunk(c, carry):
        tb = base + c * CH
        pltpu.sync_copy(slot_hbm.at[pl.ds(tb, CH)], idx0_v)
        pltpu.sync_copy(slot_hbm.at[pl.ds(T + tb, CH)], idx1_v)
        pltpu.sync_copy(x_hbm.at[pl.ds(tb, CH), :], rows_v)
        cp0 = pltpu.async_copy(rows_v, xg_hbm.at[idx0_v], s0)
        cp1 = pltpu.async_copy(rows_v, xg_hbm.at[idx1_v], s1)
        cp0.wait()
        cp1.wait()
        return carry

    lax.fori_loop(0, NCH, chunk, 0)


def _grouped_body(be_ref, na_ref, xg_ref, w1_ref, b1_ref, w2_ref, b2_ref,
                  og_ref):
    b = pl.program_id(0)

    @pl.when(b < na_ref[0])
    def _():
        h = jnp.dot(xg_ref[...], w1_ref[0],
                    preferred_element_type=jnp.float32) + b1_ref[0]
        h = _gelu(h)
        og_ref[...] = jnp.dot(h, w2_ref[0],
                              preferred_element_type=jnp.float32) + b2_ref[0]


def _grouped(be, nact, xg, rw1, rb1, rw2, rb2):
    def clamp(b, be_s, na_s):
        return jnp.minimum(b, na_s[0] - 1)

    return pl.pallas_call(
        _grouped_body,
        grid_spec=pltpu.PrefetchScalarGridSpec(
            num_scalar_prefetch=2,
            grid=(NBMAX,),
            in_specs=[
                pl.BlockSpec((BM, D), lambda b, be_s, na_s: (clamp(b, be_s, na_s), 0)),
                pl.BlockSpec((1, D, FF), lambda b, be_s, na_s: (be_s[b], 0, 0)),
                pl.BlockSpec((1, 1, FF), lambda b, be_s, na_s: (be_s[b], 0, 0)),
                pl.BlockSpec((1, FF, D), lambda b, be_s, na_s: (be_s[b], 0, 0)),
                pl.BlockSpec((1, 1, D), lambda b, be_s, na_s: (be_s[b], 0, 0)),
            ],
            out_specs=pl.BlockSpec(
                (BM, D), lambda b, be_s, na_s: (clamp(b, be_s, na_s), 0)),
        ),
        out_shape=jax.ShapeDtypeStruct((PPAD, D), jnp.float32),
        compiler_params=pltpu.CompilerParams(
            dimension_semantics=("arbitrary",)),
    )(be, nact, xg, rw1, rb1, rw2, rb2)


DH = D // 2
_UNROLL = 8


@functools.lru_cache(maxsize=None)
def _build_sc_combine():
    return functools.partial(
        pl.kernel,
        mesh=_sc_mesh(),
        out_type=jax.ShapeDtypeStruct((T, D), jnp.float32),
        scratch_types=[
            pltpu.VMEM((CH, DH), jnp.float32),
            pltpu.VMEM((CH, DH), jnp.float32),
            pltpu.VMEM((CH, DH), jnp.float32),
            pltpu.VMEM((CH,), jnp.int32),
            pltpu.VMEM((CH,), jnp.int32),
            pltpu.VMEM((TPW, 16), jnp.float32),
            pltpu.VMEM((TPW, 16), jnp.float32),
            pltpu.SemaphoreType.DMA,
            pltpu.SemaphoreType.DMA,
            pltpu.SemaphoreType.DMA,
        ],
    )(_sc_combine_body)


def _sc_combine_body(og2_hbm, so_hbm, slot2_hbm, p_hbm, out_hbm,
                     a_v, b_v, s_v, idx0_v, idx1_v, p0_v, p1_v, sa, sb, ss):
    wid = lax.axis_index("s") * 2 + lax.axis_index("c")
    base = wid * TPW
    pltpu.sync_copy(p_hbm.at[pl.ds(base, TPW), :], p0_v)
    pltpu.sync_copy(p_hbm.at[pl.ds(T + base, TPW), :], p1_v)

    def chunk(c, carry):
        tb = base + c * CH
        for h in range(2):
            pltpu.sync_copy(slot2_hbm.at[pl.ds(h * NP + tb, CH)], idx0_v)
            pltpu.sync_copy(slot2_hbm.at[pl.ds(h * NP + T + tb, CH)], idx1_v)
            cpa = pltpu.async_copy(og2_hbm.at[idx0_v], a_v, sa)
            cpb = pltpu.async_copy(og2_hbm.at[idx1_v], b_v, sb)
            cps = pltpu.async_copy(
                so_hbm.at[pl.ds(tb, CH), pl.ds(h * DH, DH)], s_v, ss)
            cpa.wait()
            cpb.wait()
            cps.wait()

            def tok(i, carry2):
                m0 = p0_v[c * CH + i, :]
                m1 = p1_v[c * CH + i, :]

                def col(j, carry3):
                    for u in range(_UNROLL):
                        sl = pl.ds(j * (16 * _UNROLL) + u * 16, 16)
                        s_v[i, sl] = (s_v[i, sl] + m0 * a_v[i, sl]
                                      + m1 * b_v[i, sl])
                    return carry3

                return lax.fori_loop(0, DH // (16 * _UNROLL), col, carry2)

            lax.fori_loop(0, CH, tok, 0)
            pltpu.sync_copy(s_v, out_hbm.at[pl.ds(tb, CH), pl.ds(h * DH, DH)])
        return carry

    lax.fori_loop(0, NCH, chunk, 0)


def kernel(x, gate_w, gate_b, shared_w1, shared_b1, shared_w2, shared_b2,
           routed_w1, routed_b1, routed_w2, routed_b2):
    gb = gate_b.reshape(1, NE)
    w1c = jnp.concatenate([shared_w1[0], shared_w1[1]], axis=1)
    b1c = jnp.concatenate([shared_b1[0], shared_b1[1]]).reshape(1, FF2)
    w2c = jnp.concatenate([shared_w2[0], shared_w2[1]], axis=0)
    b2c = (shared_b2[0] + shared_b2[1]).reshape(1, D)
    rb1 = routed_b1.reshape(NE, 1, FF)
    rb2 = routed_b2.reshape(NE, 1, D)

    oh0, oh1, p0, p1, so = _gate_shared(x, gate_w, gb, w1c, b1c, w2c, b2c)
    slot, slot2, be, nact = _plan(oh0, oh1)
    slot_f = slot.reshape(NP)
    p_f = jnp.broadcast_to(
        jnp.concatenate([p0, p1], axis=0), (NP, 16))
    xg = _build_sc_dispatch()(x, slot_f)
    og = _grouped(be.reshape(NBMAX), nact.reshape(1), xg,
                  routed_w1, rb1, routed_w2, rb2)
    out = _build_sc_combine()(og.reshape(PPAD * 2, DH), so,
                              slot2.reshape(2 * NP), p_f)
    return out


# speedup vs baseline: 1.2383x; 1.2383x over previous
import functools

import jax
import jax.numpy as jnp
from jax import lax
from jax.experimental import pallas as pl
from jax.experimental.pallas import tpu as pltpu
from jax.experimental.pallas import tpu_sc as plsc

D = 5120
FF = 384
NE = 32
NS = 2
TOPK = 2
T = 2048

TB = 256
NTB = T // TB
BM = 128
NP = T * TOPK
NBMAX = NP // BM + NE
PPAD = NBMAX * BM

NW = 32
TPW = T // NW
CH = 8
NCH = TPW // CH
FF2 = NS * FF

_INV_SQRT2 = 0.7071067811865476


def _erf(x):
    s = jnp.sign(x)
    a = jnp.abs(x)
    t = 1.0 / (1.0 + 0.3275911 * a)
    poly = t * (0.254829592 + t * (-0.284496736 + t * (1.421413741 + t * (
        -1.453152027 + t * 1.061405429))))
    return s * (1.0 - poly * jnp.exp(-a * a))


def _gelu(x):
    return 0.5 * x * (1.0 + _erf(x * _INV_SQRT2))


def _gate_shared_body(xb, gw, gb, w1, b1, w2, b2,
                      oh0, oh1, p0, p1, so):
    xv = xb[...]
    logits = jnp.dot(xv, gw[...], preferred_element_type=jnp.float32) + gb[...]
    m = jnp.max(logits, axis=1, keepdims=True)
    ex = jnp.exp(logits - m)
    probs = ex / jnp.sum(ex, axis=1, keepdims=True)
    lane = lax.broadcasted_iota(jnp.int32, (TB, NE), 1)
    m1 = jnp.max(probs, axis=1, keepdims=True)
    a1 = jnp.min(jnp.where(probs == m1, lane, NE), axis=1, keepdims=True)
    sel1 = lane == a1
    pm = jnp.where(sel1, -jnp.inf, probs)
    m2 = jnp.max(pm, axis=1, keepdims=True)
    a2 = jnp.min(jnp.where(pm == m2, lane, NE), axis=1, keepdims=True)
    sel2 = lane == a2
    oh0[...] = sel1.astype(jnp.float32)
    oh1[...] = sel2.astype(jnp.float32)
    p0[...] = m1
    p1[...] = m2

    h = jnp.dot(xv, w1[...], preferred_element_type=jnp.float32) + b1[...]
    h = _gelu(h)
    so[...] = jnp.dot(h, w2[...], preferred_element_type=jnp.float32) + b2[...]


def _gate_shared(x, gate_w, gb, w1c, b1c, w2c, b2c):
    return pl.pallas_call(
        _gate_shared_body,
        grid=(NTB,),
        in_specs=[
            pl.BlockSpec((TB, D), lambda t: (t, 0)),
            pl.BlockSpec((D, NE), lambda t: (0, 0)),
            pl.BlockSpec((1, NE), lambda t: (0, 0)),
            pl.BlockSpec((D, FF2), lambda t: (0, 0)),
            pl.BlockSpec((1, FF2), lambda t: (0, 0)),
            pl.BlockSpec((FF2, D), lambda t: (0, 0)),
            pl.BlockSpec((1, D), lambda t: (0, 0)),
        ],
        out_specs=[
            pl.BlockSpec((TB, NE), lambda t: (t, 0)),
            pl.BlockSpec((TB, NE), lambda t: (t, 0)),
            pl.BlockSpec((TB, 1), lambda t: (t, 0)),
            pl.BlockSpec((TB, 1), lambda t: (t, 0)),
            pl.BlockSpec((TB, D), lambda t: (t, 0)),
        ],
        out_shape=[
            jax.ShapeDtypeStruct((T, NE), jnp.float32),
            jax.ShapeDtypeStruct((T, NE), jnp.float32),
            jax.ShapeDtypeStruct((T, 1), jnp.float32),
            jax.ShapeDtypeStruct((T, 1), jnp.float32),
            jax.ShapeDtypeStruct((T, D), jnp.float32),
        ],
        compiler_params=pltpu.CompilerParams(
            dimension_semantics=("arbitrary",)),
    )(x, gate_w, gb, w1c, b1c, w2c, b2c)


def _cumsum0(a, n):
    s = 1
    while s < n:
        shifted = jnp.concatenate(
            [jnp.zeros((s, NE), jnp.float32), a[:-s, :]], axis=0)
        a = a + shifted
        s *= 2
    return a


def _plan_body(oh0_ref, oh1_ref, slot_ref, s2_ref, be_ref, nact_ref):
    oh0 = oh0_ref[...]
    oh1 = oh1_ref[...]
    cum0 = _cumsum0(oh0, T)
    cum1 = _cumsum0(oh1, T)
    tot0 = cum0[T - 1:T, :]
    tot1 = cum1[T - 1:T, :]
    cnt = tot0 + tot1
    nb = jnp.floor((cnt + (BM - 1)) * (1.0 / BM))
    r = lax.broadcasted_iota(jnp.int32, (NE, NE), 0)
    c = lax.broadcasted_iota(jnp.int32, (NE, NE), 1)
    ltri = (r < c).astype(jnp.float32)
    bstart = jnp.dot(nb, ltri, preferred_element_type=jnp.float32)
    nact = jnp.sum(nb, axis=1, keepdims=True)
    start_rows = bstart * BM
    slot0 = jnp.sum(oh0 * (start_rows + cum0 - 1.0), axis=1, keepdims=True)
    slot1 = jnp.sum(oh1 * (start_rows + tot0 + cum1 - 1.0), axis=1,
                    keepdims=True)
    slot_ref[0:T, :] = slot0.astype(jnp.int32)
    slot_ref[T:NP, :] = slot1.astype(jnp.int32)
    s2_ref[0:T, :] = slot0.astype(jnp.int32)
    s2_ref[T:NP, :] = slot1.astype(jnp.int32)
    s2_ref[NP:NP + T, :] = (slot0 + float(PPAD)).astype(jnp.int32)
    s2_ref[NP + T:2 * NP, :] = (slot1 + float(PPAD)).astype(jnp.int32)
    bio = lax.broadcasted_iota(jnp.int32, (NBMAX, 1), 0).astype(jnp.float32)
    bclamp = jnp.minimum(bio, nact - 1.0)
    be = jnp.sum(jnp.where(bstart <= bclamp, 1.0, 0.0), axis=1,
                 keepdims=True) - 1.0
    be_ref[...] = be.astype(jnp.int32)
    nact_ref[...] = nact.astype(jnp.int32)


def _plan(oh0, oh1):
    return pl.pallas_call(
        _plan_body,
        grid=(1,),
        in_specs=[
            pl.BlockSpec((T, NE), lambda i: (0, 0)),
            pl.BlockSpec((T, NE), lambda i: (0, 0)),
        ],
        out_specs=[
            pl.BlockSpec((NP, 1), lambda i: (0, 0)),
            pl.BlockSpec((2 * NP, 1), lambda i: (0, 0)),
            pl.BlockSpec((NBMAX, 1), lambda i: (0, 0)),
            pl.BlockSpec((1, 1), lambda i: (0, 0)),
        ],
        out_shape=[
            jax.ShapeDtypeStruct((NP, 1), jnp.int32),
            jax.ShapeDtypeStruct((2 * NP, 1), jnp.int32),
            jax.ShapeDtypeStruct((NBMAX, 1), jnp.int32),
            jax.ShapeDtypeStruct((1, 1), jnp.int32),
        ],
    )(oh0, oh1)


def _sc_mesh():
    return plsc.VectorSubcoreMesh(core_axis_name="c", subcore_axis_name="s",
                                  num_cores=2, num_subcores=16)


@functools.lru_cache(maxsize=None)
def _build_sc_dispatch():
    return functools.partial(
        pl.kernel,
        mesh=_sc_mesh(),
        out_type=jax.ShapeDtypeStruct((PPAD, D), jnp.float32),
        scratch_types=[
            pltpu.VMEM((CH, D), jnp.float32),
            pltpu.VMEM((CH,), jnp.int32),
            pltpu.VMEM((CH,), jnp.int32),
            pltpu.SemaphoreType.DMA,
            pltpu.SemaphoreType.DMA,
        ],
    )(_sc_dispatch_body)


def _sc_dispatch_body(x_hbm, slot_hbm, xg_hbm, rows_v, idx0_v, idx1_v, s0, s1):
    wid = lax.axis_index("s") * 2 + lax.axis_index("c")
    base = wid * TPW

    def chunk(c, carry):
        tb = base + c * CH
        pltpu.sync_copy(slot_hbm.at[pl.ds(tb, CH)], idx0_v)
        pltpu.sync_copy(slot_hbm.at[pl.ds(T + tb, CH)], idx1_v)
        pltpu.sync_copy(x_hbm.at[pl.ds(tb, CH), :], rows_v)
        cp0 = pltpu.async_copy(rows_v, xg_hbm.at[idx0_v], s0)
        cp1 = pltpu.async_copy(rows_v, xg_hbm.at[idx1_v], s1)
        cp0.wait()
        cp1.wait()
        return carry

    lax.fori_loop(0, NCH, chunk, 0)


def _grouped_body(be_ref, na_ref, xg_ref, w1_ref, b1_ref, w2_ref, b2_ref,
                  og_ref):
    b = pl.program_id(0)

    @pl.when(b < na_ref[0])
    def _():
        h = jnp.dot(xg_ref[...], w1_ref[0],
                    preferred_element_type=jnp.float32) + b1_ref[0]
        h = _gelu(h)
        res = jnp.dot(h, w2_ref[0],
                      preferred_element_type=jnp.float32) + b2_ref[0]
        og_ref[0] = res[:, 0:DH]
        og_ref[1] = res[:, DH:D]


def _grouped(be, nact, xg, rw1, rb1, rw2, rb2):
    def clamp(b, be_s, na_s):
        return jnp.minimum(b, na_s[0] - 1)

    return pl.pallas_call(
        _grouped_body,
        grid_spec=pltpu.PrefetchScalarGridSpec(
            num_scalar_prefetch=2,
            grid=(NBMAX,),
            in_specs=[
                pl.BlockSpec((BM, D), lambda b, be_s, na_s: (clamp(b, be_s, na_s), 0)),
                pl.BlockSpec((1, D, FF), lambda b, be_s, na_s: (be_s[b], 0, 0)),
                pl.BlockSpec((1, 1, FF), lambda b, be_s, na_s: (be_s[b], 0, 0)),
                pl.BlockSpec((1, FF, D), lambda b, be_s, na_s: (be_s[b], 0, 0)),
                pl.BlockSpec((1, 1, D), lambda b, be_s, na_s: (be_s[b], 0, 0)),
            ],
            out_specs=pl.BlockSpec(
                (2, BM, DH),
                lambda b, be_s, na_s: (0, clamp(b, be_s, na_s), 0)),
        ),
        out_shape=jax.ShapeDtypeStruct((2, PPAD, DH), jnp.float32),
        compiler_params=pltpu.CompilerParams(
            dimension_semantics=("arbitrary",)),
    )(be, nact, xg, rw1, rb1, rw2, rb2)


DH = D // 2
_UNROLL = 8


@functools.lru_cache(maxsize=None)
def _build_sc_combine():
    return functools.partial(
        pl.kernel,
        mesh=_sc_mesh(),
        out_type=jax.ShapeDtypeStruct((T, D), jnp.float32),
        scratch_types=[
            pltpu.VMEM((CH, DH), jnp.float32),
            pltpu.VMEM((CH, DH), jnp.float32),
            pltpu.VMEM((CH, DH), jnp.float32),
            pltpu.VMEM((CH,), jnp.int32),
            pltpu.VMEM((CH,), jnp.int32),
            pltpu.VMEM((TPW, 16), jnp.float32),
            pltpu.VMEM((TPW, 16), jnp.float32),
            pltpu.SemaphoreType.DMA,
            pltpu.SemaphoreType.DMA,
            pltpu.SemaphoreType.DMA,
        ],
    )(_sc_combine_body)


def _sc_combine_body(og2_hbm, so_hbm, slot2_hbm, p_hbm, out_hbm,
                     a_v, b_v, s_v, idx0_v, idx1_v, p0_v, p1_v, sa, sb, ss):
    wid = lax.axis_index("s") * 2 + lax.axis_index("c")
    base = wid * TPW
    pltpu.sync_copy(p_hbm.at[pl.ds(base, TPW), :], p0_v)
    pltpu.sync_copy(p_hbm.at[pl.ds(T + base, TPW), :], p1_v)

    def chunk(c, carry):
        tb = base + c * CH
        for h in range(2):
            pltpu.sync_copy(slot2_hbm.at[pl.ds(h * NP + tb, CH)], idx0_v)
            pltpu.sync_copy(slot2_hbm.at[pl.ds(h * NP + T + tb, CH)], idx1_v)
            cpa = pltpu.async_copy(og2_hbm.at[idx0_v], a_v, sa)
            cpb = pltpu.async_copy(og2_hbm.at[idx1_v], b_v, sb)
            cps = pltpu.async_copy(
                so_hbm.at[pl.ds(tb, CH), pl.ds(h * DH, DH)], s_v, ss)
            cpa.wait()
            cpb.wait()
            cps.wait()

            def tok(i, carry2):
                m0 = p0_v[c * CH + i, :]
                m1 = p1_v[c * CH + i, :]

                def col(j, carry3):
                    for u in range(_UNROLL):
                        sl = pl.ds(j * (16 * _UNROLL) + u * 16, 16)
                        s_v[i, sl] = (s_v[i, sl] + m0 * a_v[i, sl]
                                      + m1 * b_v[i, sl])
                    return carry3

                return lax.fori_loop(0, DH // (16 * _UNROLL), col, carry2)

            lax.fori_loop(0, CH, tok, 0)
            pltpu.sync_copy(s_v, out_hbm.at[pl.ds(tb, CH), pl.ds(h * DH, DH)])
        return carry

    lax.fori_loop(0, NCH, chunk, 0)


def kernel(x, gate_w, gate_b, shared_w1, shared_b1, shared_w2, shared_b2,
           routed_w1, routed_b1, routed_w2, routed_b2):
    gb = gate_b.reshape(1, NE)
    w1c = jnp.concatenate([shared_w1[0], shared_w1[1]], axis=1)
    b1c = jnp.concatenate([shared_b1[0], shared_b1[1]]).reshape(1, FF2)
    w2c = jnp.concatenate([shared_w2[0], shared_w2[1]], axis=0)
    b2c = (shared_b2[0] + shared_b2[1]).reshape(1, D)
    rb1 = routed_b1.reshape(NE, 1, FF)
    rb2 = routed_b2.reshape(NE, 1, D)

    oh0, oh1, p0, p1, so = _gate_shared(x, gate_w, gb, w1c, b1c, w2c, b2c)
    slot, slot2, be, nact = _plan(oh0, oh1)
    slot_f = slot.reshape(NP)
    p_f = jnp.broadcast_to(
        jnp.concatenate([p0, p1], axis=0), (NP, 16))
    xg = _build_sc_dispatch()(x, slot_f)
    og = _grouped(be.reshape(NBMAX), nact.reshape(1), xg,
                  routed_w1, rb1, routed_w2, rb2)
    out = _build_sc_combine()(og.reshape(2 * PPAD, DH), so,
                              slot2.reshape(2 * NP), p_f)
    return out
